# baseline (speedup 1.0000x reference)
import functools

import jax
import jax.numpy as jnp
from jax import lax
from jax.experimental import pallas as pl
from jax.experimental.pallas import tpu as pltpu
from jax.experimental.pallas import tpu_sc as plsc

_NC = 2
_NS = 16
_NW = _NC * _NS

_GDN = lax.GatherDimensionNumbers(
    offset_dims=(), collapsed_slice_dims=(0,), start_index_map=(0,))


def _lane_bcast(v, lane):
    idx = jnp.full((16,), lane, jnp.int32)
    return lax.gather(v, idx[:, None], dimension_numbers=_GDN,
                      slice_sizes=(1,),
                      mode=lax.GatherScatterMode.PROMISE_IN_BOUNDS)


def _scale_body(t_ref, s_ref):
    t = t_ref[...]
    ss = jnp.sum(t * t, axis=1, keepdims=True)
    norm = jnp.sqrt(ss)
    s_ref[...] = jnp.where(norm > 1.0, 1.0 / (norm + 1e-7), 1.0)


@functools.lru_cache(maxsize=None)
def _make_scale(V, D, BV):
    return pl.pallas_call(
        _scale_body,
        grid=(V // BV,),
        in_specs=[pl.BlockSpec((BV, D), lambda i: (i, 0))],
        out_specs=pl.BlockSpec((BV, 1), lambda i: (i, 0)),
        out_shape=jax.ShapeDtypeStruct((V, 1), jnp.float32),
    )


@functools.lru_cache(maxsize=None)
def _make_pool_kernel(B, L, D, V):
    CB = 16
    RPC = CB * L
    PW = B // _NW
    NCH = PW // CB
    KD = D // 16
    mesh = plsc.VectorSubcoreMesh(core_axis_name="c", subcore_axis_name="s")

    @functools.partial(
        pl.kernel,
        mesh=mesh,
        out_type=jax.ShapeDtypeStruct((B, D), jnp.float32),
        scratch_types=[
            pltpu.VMEM((RPC,), jnp.int32),
            pltpu.VMEM((RPC,), jnp.int32),
            pltpu.VMEM((RPC, D), jnp.float32),
            pltpu.VMEM((RPC, D), jnp.float32),
            pltpu.VMEM((RPC + 16,), jnp.float32),
            pltpu.VMEM((RPC + 16,), jnp.float32),
            pltpu.VMEM((CB, D), jnp.float32),
            pltpu.SemaphoreType.DMA,
            pltpu.SemaphoreType.DMA,
        ],
    )
    def pool(x_hbm, table_hbm, scale_hbm, out_hbm, idx_v0, idx_v1,
             rows_v0, rows_v1, scl_v0, scl_v1, pooled_v, sem0, sem1):
        wid = lax.axis_index("s") * _NC + lax.axis_index("c")
        base_b0 = wid * PW

        def start_fetch(ci, idx_v, rows_v, scl_v, sem):
            base_r = (base_b0 + ci * CB) * L
            pltpu.sync_copy(x_hbm.at[pl.ds(base_r, RPC)], idx_v)
            pltpu.async_copy(table_hbm.at[idx_v], rows_v, sem)
            pltpu.async_copy(scale_hbm.at[idx_v], scl_v.at[pl.ds(0, RPC)], sem)

        def wait_fetch(idx_v, rows_v, scl_v, sem):
            pltpu.make_async_copy(table_hbm.at[idx_v], rows_v, sem).wait()
            pltpu.make_async_copy(
                scale_hbm.at[idx_v], scl_v.at[pl.ds(0, RPC)], sem).wait()

        def compute_chunk(ci, rows_v, scl_v):
            def batch_body(j, carry):
                r0 = j * L
                sc_lo = scl_v[pl.ds(r0, 16)]
                sc_hi = scl_v[pl.ds(r0 + 16, 16)]
                accs = [jnp.zeros((16,), jnp.float32)] * KD
                for l in range(L):
                    r = r0 + l
                    sc = (_lane_bcast(sc_lo, l) if l < 16
                          else _lane_bcast(sc_hi, l - 16))
                    vs = [rows_v[r, pl.ds(16 * k, 16)] for k in range(KD)]
                    accs = [a + sc * v for a, v in zip(accs, vs)]
                inv = jnp.float32(1.0 / L)
                for k in range(KD):
                    pooled_v[j, pl.ds(16 * k, 16)] = accs[k] * inv
                return carry

            lax.fori_loop(0, CB, batch_body, 0)
            pltpu.sync_copy(pooled_v, out_hbm.at[pl.ds(base_b0 + ci * CB, CB)])

        start_fetch(0, idx_v0, rows_v0, scl_v0, sem0)

        def pair_body(p, carry):
            ci0 = 2 * p
            wait_fetch(idx_v0, rows_v0, scl_v0, sem0)
            start_fetch(ci0 + 1, idx_v1, rows_v1, scl_v1, sem1)
            compute_chunk(ci0, rows_v0, scl_v0)
            wait_fetch(idx_v1, rows_v1, scl_v1, sem1)

            @pl.when(p + 1 < NCH // 2)
            def _():
                start_fetch(ci0 + 2, idx_v0, rows_v0, scl_v0, sem0)

            compute_chunk(ci0 + 1, rows_v1, scl_v1)
            return carry

        lax.fori_loop(0, NCH // 2, pair_body, 0)

    return pool


def _mlp_body(xe_ref, w1_ref, b1_ref, w2_ref, b2_ref, fc1_ref, fc2_ref, pred_ref):
    x = xe_ref[...]
    h = lax.dot_general(x, w1_ref[...], (((1,), (1,)), ((), ())),
                        preferred_element_type=jnp.float32)
    h = jnp.maximum(h + b1_ref[...], 0.0)
    fc1_ref[...] = h
    z = jnp.sum(h * w2_ref[...], axis=1, keepdims=True) + b2_ref[...]
    fc2_ref[...] = z
    pred_ref[...] = 1.0 / (1.0 + jnp.exp(-z))


@functools.lru_cache(maxsize=None)
def _make_mlp(B, D, H, BT):
    grid = (B // BT,)
    return pl.pallas_call(
        _mlp_body,
        grid=grid,
        in_specs=[
            pl.BlockSpec((BT, D), lambda i: (i, 0)),
            pl.BlockSpec((H, D), lambda i: (0, 0)),
            pl.BlockSpec((1, H), lambda i: (0, 0)),
            pl.BlockSpec((1, H), lambda i: (0, 0)),
            pl.BlockSpec((1, 1), lambda i: (0, 0)),
        ],
        out_specs=[
            pl.BlockSpec((BT, H), lambda i: (i, 0)),
            pl.BlockSpec((BT, 1), lambda i: (i, 0)),
            pl.BlockSpec((BT, 1), lambda i: (i, 0)),
        ],
        out_shape=[
            jax.ShapeDtypeStruct((B, H), jnp.float32),
            jax.ShapeDtypeStruct((B, 1), jnp.float32),
            jax.ShapeDtypeStruct((B, 1), jnp.float32),
        ],
    )


def kernel(x, table, W1, b1, W2, b2):
    B, L = x.shape
    V, D = table.shape
    H = W1.shape[0]
    x_flat = x.reshape(B * L).astype(jnp.int32)
    scale = _make_scale(V, D, 2000)(table).reshape(V)
    x_embed = _make_pool_kernel(B, L, D, V)(x_flat, table, scale)
    fc1, fc2, pred = _make_mlp(B, D, H, 1024)(
        x_embed, W1, b1.reshape(1, H), W2, b2.reshape(1, 1))
    return fc1, fc2, pred


# speedup vs baseline: 1.1036x; 1.1036x over previous
import functools

import jax
import jax.numpy as jnp
import numpy as np
from jax import lax
from jax.experimental import pallas as pl
from jax.experimental.pallas import tpu as pltpu
from jax.experimental.pallas import tpu_sc as plsc

_NC = 2
_NS = 16
_NW = _NC * _NS


def _deinterleave_perm(D):
    perm = np.empty(D, np.int32)
    half = D // 2
    for k in range(D // 32):
        perm[32 * k:32 * k + 16] = 16 * k + np.arange(16)
        perm[32 * k + 16:32 * k + 32] = half + 16 * k + np.arange(16)
    return perm


def _bf16_bits(x):
    bits = lax.bitcast_convert_type(x, jnp.uint32)
    lsb = (bits >> 16) & jnp.uint32(1)
    r = bits + jnp.uint32(0x7FFF) + lsb
    return r >> 16


def _scale_body(t_ref, o_ref):
    t = t_ref[...]
    ss = jnp.sum(t * t, axis=1, keepdims=True)
    norm = jnp.sqrt(ss)
    sc = jnp.where(norm > 1.0, 1.0 / (norm + 1e-7), 1.0)
    ts = t * sc
    half = t.shape[1] // 2
    ra = _bf16_bits(ts[:, :half])
    rb = _bf16_bits(ts[:, half:])
    word = ra | (rb << 16)
    o_ref[...] = lax.bitcast_convert_type(word, jnp.int32)


@functools.lru_cache(maxsize=None)
def _make_scale(V, D, BV):
    return pl.pallas_call(
        _scale_body,
        grid=(V // BV,),
        in_specs=[pl.BlockSpec((BV, D), lambda i: (i, 0))],
        out_specs=pl.BlockSpec((BV, D // 2), lambda i: (i, 0)),
        out_shape=jax.ShapeDtypeStruct((V, D // 2), jnp.int32),
    )


@functools.lru_cache(maxsize=None)
def _make_pool_kernel(B, L, D, V):
    CB = 32
    RPC = CB * L
    PW = B // _NW
    NCH = PW // CB
    KD = D // 32
    DW = D // 2
    mesh = plsc.VectorSubcoreMesh(core_axis_name="c", subcore_axis_name="s")

    @functools.partial(
        pl.kernel,
        mesh=mesh,
        compiler_params=pltpu.CompilerParams(
            needs_layout_passes=False, use_tc_tiling_on_sc=False),
        out_type=jax.ShapeDtypeStruct((B, D), jnp.float32),
        scratch_types=[
            pltpu.VMEM((RPC,), jnp.int32),
            pltpu.VMEM((RPC,), jnp.int32),
            pltpu.VMEM((RPC, DW), jnp.int32),
            pltpu.VMEM((RPC, DW), jnp.int32),
            pltpu.VMEM((CB, D), jnp.float32),
            pltpu.SemaphoreType.DMA,
            pltpu.SemaphoreType.DMA,
        ],
    )
    def pool(x_hbm, table_hbm, out_hbm, idx_v0, idx_v1,
             rows_v0, rows_v1, pooled_v, sem0, sem1):
        wid = lax.axis_index("s") * _NC + lax.axis_index("c")
        base_b0 = wid * PW

        def start_fetch(ci, idx_v, rows_v, sem):
            base_r = (base_b0 + ci * CB) * L
            pltpu.sync_copy(x_hbm.at[pl.ds(base_r, RPC)], idx_v)
            pltpu.async_copy(table_hbm.at[idx_v], rows_v, sem)

        def wait_fetch(idx_v, rows_v, sem):
            pltpu.make_async_copy(table_hbm.at[idx_v], rows_v, sem).wait()

        def compute_chunk(ci, rows_v):
            def batch_body(j, carry):
                r0 = j * L
                acca = [jnp.zeros((16,), jnp.float32)] * KD
                accb = [jnp.zeros((16,), jnp.float32)] * KD
                for l in range(L):
                    r = r0 + l
                    for k in range(KD):
                        v = rows_v[r, pl.ds(16 * k, 16)]
                        vbf = plsc.bitcast(v, jnp.bfloat16)
                        a, b = plsc.unpack(vbf, format=plsc.PackFormat.INTERLEAVED)
                        acca[k] = acca[k] + a
                        accb[k] = accb[k] + b
                inv = jnp.float32(1.0 / L)
                for k in range(KD):
                    pooled_v[j, pl.ds(32 * k, 16)] = acca[k] * inv
                    pooled_v[j, pl.ds(32 * k + 16, 16)] = accb[k] * inv
                return carry

            lax.fori_loop(0, CB, batch_body, 0)
            pltpu.sync_copy(pooled_v, out_hbm.at[pl.ds(base_b0 + ci * CB, CB)])

        start_fetch(0, idx_v0, rows_v0, sem0)

        def pair_body(p, carry):
            ci0 = 2 * p
            wait_fetch(idx_v0, rows_v0, sem0)
            start_fetch(ci0 + 1, idx_v1, rows_v1, sem1)
            compute_chunk(ci0, rows_v0)
            wait_fetch(idx_v1, rows_v1, sem1)

            @pl.when(p + 1 < NCH // 2)
            def _():
                start_fetch(ci0 + 2, idx_v0, rows_v0, sem0)

            compute_chunk(ci0 + 1, rows_v1)
            return carry

        lax.fori_loop(0, NCH // 2, pair_body, 0)

    return pool


def _mlp_body(xe_ref, w1_ref, b1_ref, w2_ref, b2_ref, fc1_ref, fc2_ref, pred_ref):
    x = xe_ref[...]
    h = lax.dot_general(x, w1_ref[...], (((1,), (1,)), ((), ())),
                        preferred_element_type=jnp.float32)
    h = jnp.maximum(h + b1_ref[...], 0.0)
    fc1_ref[...] = h
    z = jnp.sum(h * w2_ref[...], axis=1, keepdims=True) + b2_ref[...]
    fc2_ref[...] = z
    pred_ref[...] = 1.0 / (1.0 + jnp.exp(-z))


@functools.lru_cache(maxsize=None)
def _make_mlp(B, D, H, BT):
    grid = (B // BT,)
    return pl.pallas_call(
        _mlp_body,
        grid=grid,
        in_specs=[
            pl.BlockSpec((BT, D), lambda i: (i, 0)),
            pl.BlockSpec((H, D), lambda i: (0, 0)),
            pl.BlockSpec((1, H), lambda i: (0, 0)),
            pl.BlockSpec((1, H), lambda i: (0, 0)),
            pl.BlockSpec((1, 1), lambda i: (0, 0)),
        ],
        out_specs=[
            pl.BlockSpec((BT, H), lambda i: (i, 0)),
            pl.BlockSpec((BT, 1), lambda i: (i, 0)),
            pl.BlockSpec((BT, 1), lambda i: (i, 0)),
        ],
        out_shape=[
            jax.ShapeDtypeStruct((B, H), jnp.float32),
            jax.ShapeDtypeStruct((B, 1), jnp.float32),
            jax.ShapeDtypeStruct((B, 1), jnp.float32),
        ],
    )


def kernel(x, table, W1, b1, W2, b2):
    B, L = x.shape
    V, D = table.shape
    H = W1.shape[0]
    x_flat = x.reshape(B * L).astype(jnp.int32)
    scaled_tab = _make_scale(V, D, 2000)(table)
    x_embed = _make_pool_kernel(B, L, D, V)(x_flat, scaled_tab)
    W1p = W1[:, _deinterleave_perm(D)]
    fc1, fc2, pred = _make_mlp(B, D, H, 1024)(
        x_embed, W1p, b1.reshape(1, H), W2, b2.reshape(1, 1))
    return fc1, fc2, pred
